# baseline (speedup 1.0000x reference)
import jax
import jax.numpy as jnp
from jax.experimental import pallas as pl
from jax.experimental.pallas import tpu as pltpu

B, T, C, L = 32, 500, 256, 50
S = 2 * L + 1
SP = 128
T_BLK = 100
NT = T // T_BLK
NORM = 10
DEN_SCALE = 1.0
NEG = -1e30


def _lp_block(x_ref, lp3_ref, sl):
    x = x_ref[...]
    m = jnp.max(x, axis=2, keepdims=True)
    e = jnp.exp(x - m)
    lse = m + jnp.log(jnp.sum(e, axis=2, keepdims=True))
    lp3_ref[sl] = (x - lse).astype(jnp.bfloat16)


def _emissions(lp3_ref, sl, oh_ref, pemit2_ref, psl, bs):
    for b in bs:
        pemit2_ref[psl, :, b, :] = jnp.dot(
            lp3_ref[sl, :, b, :], oh_ref[b],
            preferred_element_type=jnp.float32)


def _fwd_kernel(x_ref, ext_ref, skip_ref, dent_ref, frames_ref,
                out_s_ref, out_tf_ref, out_af_ref,
                p_ref, oh_ref, lp3_ref, pemit2_ref,
                z_ref, lz_ref, na_ref):
    i = pl.program_id(0)

    @pl.when(i == 0)
    def _init():
        dt = dent_ref[...]
        m = jnp.max(dt, axis=1, keepdims=True)
        e = jnp.exp(dt - m)
        p_ref[...] = (e / jnp.sum(e, axis=1, keepdims=True)).astype(jnp.bfloat16)
        ids = jax.lax.broadcasted_iota(jnp.int32, (B, C, SP), 1)
        oh_ref[...] = (ids == ext_ref[...][:, None, :]).astype(jnp.bfloat16)
        _lp_block(x_ref, lp3_ref, 0)

    @pl.when(i == 1)
    def _warm():
        _lp_block(x_ref, lp3_ref, 1)
        _emissions(lp3_ref, 0, oh_ref, pemit2_ref, 0, range(B))

    @pl.when(i >= 2)
    def _steady():
        sl_a = jax.lax.rem(i, 3)
        sl_b = jax.lax.rem(i + 2, 3)
        sl_c = jax.lax.rem(i + 1, 3)
        psl_b = jax.lax.rem(i + 1, 2)
        psl_c = jax.lax.rem(i, 2)
        first = i == 2
        skip = skip_ref[...] > 0
        lane = jax.lax.broadcasted_iota(jnp.int32, (B, SP), 1)
        z = z_ref[...]
        lz = jnp.where(first, 0.0, lz_ref[...])
        na = jnp.where(first, jnp.where(lane == 0, 0.0, NEG), na_ref[...])
        for t in range(T_BLK):
            lpt = lp3_ref[sl_c, t]
            pt = jnp.exp(lpt)
            et = pemit2_ref[psl_c, t]
            u = jnp.dot(z, p_ref[...],
                        preferred_element_type=jnp.float32).astype(jnp.bfloat16) * pt
            if t == 0:
                u = jnp.where(first, pt * (1.0 / C), u)
            if t % NORM == NORM - 1:
                s = jnp.sum(u.astype(jnp.float32), axis=1, keepdims=True)
                z = u * (1.0 / s).astype(jnp.bfloat16)
                lz = lz + jnp.log(s)
            else:
                z = u
            a1 = jnp.concatenate(
                [jnp.full((B, 1), NEG, jnp.float32), na[:, :-1]], axis=1)
            a2 = jnp.where(
                skip,
                jnp.concatenate([jnp.full((B, 2), NEG, jnp.float32), na[:, :-2]], axis=1),
                jnp.float32(NEG))
            mx = jnp.maximum(jnp.maximum(na, a1), a2)
            mxe = mx + et
            na = mxe + jnp.log(jnp.exp(na - mx) + jnp.exp(a1 - mx) + jnp.exp(a2 - mx))
            xs = x_ref[t]
            mm = jnp.max(xs, axis=1, keepdims=True)
            ee = jnp.exp(xs - mm)
            lse = mm + jnp.log(jnp.sum(ee, axis=1, keepdims=True))
            lp3_ref[sl_a, t] = (xs - lse).astype(jnp.bfloat16)
            if t % 3 == 0 and t // 3 < B:
                _emissions(lp3_ref, sl_b, oh_ref, pemit2_ref, psl_b, [t // 3])
        z_ref[...] = z
        lz_ref[...] = lz
        na_ref[...] = na

    @pl.when(i == NT + 1)
    def _finish():
        na = na_ref[...]
        lz = lz_ref[...]
        aL, aK = na[:, S - 1:S], na[:, S - 2:S - 1]
        m2 = jnp.maximum(aL, aK)
        num_tot = m2 + jnp.log(jnp.exp(aL - m2) + jnp.exp(aK - m2))
        ts = num_tot - DEN_SCALE * lz
        fr = frames_ref[...][:, 0:1]
        mask = jnp.isfinite(ts) & (ts > -1e20)
        out_s_ref[...] = jnp.sum(jnp.where(mask, ts, 0.0)).reshape(1, 1)
        out_tf_ref[...] = jnp.sum(jnp.where(mask, fr, 0.0)).reshape(1, 1)
        out_af_ref[...] = jnp.sum(fr).reshape(1, 1)


@jax.jit
def kernel(nnet_output, labels, supervision_segments, den_trans):
    ext = jnp.full((B, SP), -1, jnp.int32)
    ext = ext.at[:, 0:S:2].set(0)
    ext = ext.at[:, 1:S:2].set(labels)
    ext_prev2 = jnp.concatenate([jnp.full((B, 2), -1, jnp.int32), ext[:, :-2]], axis=1)
    skip = ((ext > 0) & (ext != ext_prev2)).astype(jnp.float32)
    frames = supervision_segments[:, 2].astype(jnp.float32)
    frames_b = jnp.broadcast_to(frames[:, None], (B, SP))

    xt = jnp.transpose(nnet_output, (1, 0, 2))

    grid = (NT + 2,)
    out_s, out_tf, out_af = pl.pallas_call(
        _fwd_kernel,
        grid=grid,
        in_specs=[
            pl.BlockSpec((T_BLK, B, C), lambda i: (jnp.minimum(i, NT - 1), 0, 0)),
            pl.BlockSpec((B, SP), lambda i: (0, 0)),
            pl.BlockSpec((B, SP), lambda i: (0, 0)),
            pl.BlockSpec((C, C), lambda i: (0, 0)),
            pl.BlockSpec((B, SP), lambda i: (0, 0)),
        ],
        out_specs=[
            pl.BlockSpec((1, 1), lambda i: (0, 0)),
            pl.BlockSpec((1, 1), lambda i: (0, 0)),
            pl.BlockSpec((1, 1), lambda i: (0, 0)),
        ],
        out_shape=[
            jax.ShapeDtypeStruct((1, 1), jnp.float32),
            jax.ShapeDtypeStruct((1, 1), jnp.float32),
            jax.ShapeDtypeStruct((1, 1), jnp.float32),
        ],
        scratch_shapes=[
            pltpu.VMEM((C, C), jnp.bfloat16),
            pltpu.VMEM((B, C, SP), jnp.bfloat16),
            pltpu.VMEM((3, T_BLK, B, C), jnp.bfloat16),
            pltpu.VMEM((2, T_BLK, B, SP), jnp.float32),
            pltpu.VMEM((B, C), jnp.bfloat16),
            pltpu.VMEM((B, 1), jnp.float32),
            pltpu.VMEM((B, SP), jnp.float32),
        ],
        compiler_params=pltpu.CompilerParams(
            dimension_semantics=("arbitrary",),
        ),
    )(xt, ext, skip, den_trans, frames_b)

    return out_s[0, 0], out_tf[0, 0], out_af[0, 0]


# speedup vs baseline: 1.0051x; 1.0051x over previous
import functools

import jax
import jax.numpy as jnp
from jax.experimental import pallas as pl
from jax.experimental.pallas import tpu as pltpu

B, T, C, L = 32, 500, 256, 50
S = 2 * L + 1
SP = 128
T_BLK = 100
NT = T // T_BLK
DEN_SCALE = 1.0
NEG = -1e30


def _fwd_kernel(x_ref, ext_ref, skip_ref, dent_ref, frames_ref,
                out_s_ref, out_tf_ref, out_af_ref,
                p_ref, oh_ref, probs_ref, pemit_ref,
                z_ref, lz_ref, na_ref):
    i = pl.program_id(0)

    @pl.when(i == 0)
    def _init():
        dt = dent_ref[...]
        m = jnp.max(dt, axis=1, keepdims=True)
        e = jnp.exp(dt - m)
        p_ref[...] = (e / jnp.sum(e, axis=1, keepdims=True)).astype(jnp.bfloat16)
        ids = jax.lax.broadcasted_iota(jnp.int32, (B, C, SP), 1)
        oh_ref[...] = (ids == ext_ref[...][:, None, :]).astype(jnp.bfloat16)
        z_ref[...] = jnp.zeros((B, C), jnp.float32)
        lz_ref[...] = jnp.zeros((B, 1), jnp.float32)
        lane = jax.lax.broadcasted_iota(jnp.int32, (B, SP), 1)
        na_ref[...] = jnp.where(lane == 0, 0.0, NEG)

    x = x_ref[...]
    m = jnp.max(x, axis=2, keepdims=True)
    e = jnp.exp(x - m)
    sm = jnp.sum(e, axis=2, keepdims=True)
    lp = x - m - jnp.log(sm)
    probs_ref[...] = e / sm

    lp16 = lp.astype(jnp.bfloat16)
    for b in range(B):
        pemit_ref[:, b, :] = jnp.dot(lp16[:, b, :], oh_ref[b],
                                     preferred_element_type=jnp.float32)

    P = p_ref[...]
    skip = skip_ref[...] > 0

    NORM = 10
    z = z_ref[...]
    lz = lz_ref[...]
    na = na_ref[...]
    for t in range(T_BLK):
        pt = probs_ref[t]
        et = pemit_ref[t]
        u = jnp.dot(z.astype(jnp.bfloat16), P,
                    preferred_element_type=jnp.float32) * pt
        if t == 0:
            u = jnp.where(i == 0, pt * (1.0 / C), u)
        if t % NORM == NORM - 1:
            s = jnp.sum(u, axis=1, keepdims=True)
            z = u * (1.0 / s)
            lz = lz + jnp.log(s)
        else:
            z = u
        a1 = jnp.concatenate([jnp.full((B, 1), NEG, jnp.float32), na[:, :-1]], axis=1)
        a2 = jnp.where(skip,
                       jnp.concatenate([jnp.full((B, 2), NEG, jnp.float32), na[:, :-2]], axis=1),
                       jnp.float32(NEG))
        mx = jnp.maximum(jnp.maximum(na, a1), a2)
        na = mx + jnp.log(jnp.exp(na - mx) + jnp.exp(a1 - mx) + jnp.exp(a2 - mx)) + et
    z_ref[...] = z
    lz_ref[...] = lz
    na_ref[...] = na

    @pl.when(i == NT - 1)
    def _finish():
        aL, aK = na[:, S - 1:S], na[:, S - 2:S - 1]
        m2 = jnp.maximum(aL, aK)
        num_tot = m2 + jnp.log(jnp.exp(aL - m2) + jnp.exp(aK - m2))
        den_tot = lz
        ts = num_tot - DEN_SCALE * den_tot
        fr = frames_ref[...][:, 0:1]
        mask = jnp.isfinite(ts) & (ts > -1e20)
        out_s_ref[...] = jnp.sum(jnp.where(mask, ts, 0.0)).reshape(1, 1)
        out_tf_ref[...] = jnp.sum(jnp.where(mask, fr, 0.0)).reshape(1, 1)
        out_af_ref[...] = jnp.sum(fr).reshape(1, 1)


@jax.jit
def kernel(nnet_output, labels, supervision_segments, den_trans):
    ext = jnp.full((B, SP), -1, jnp.int32)
    ext = ext.at[:, 0:S:2].set(0)
    ext = ext.at[:, 1:S:2].set(labels)
    ext_prev2 = jnp.concatenate([jnp.full((B, 2), -1, jnp.int32), ext[:, :-2]], axis=1)
    skip = ((ext > 0) & (ext != ext_prev2)).astype(jnp.float32)
    frames = supervision_segments[:, 2].astype(jnp.float32)
    frames_b = jnp.broadcast_to(frames[:, None], (B, SP))

    xt = jnp.transpose(nnet_output, (1, 0, 2))

    grid = (NT,)
    out_s, out_tf, out_af = pl.pallas_call(
        _fwd_kernel,
        grid=grid,
        in_specs=[
            pl.BlockSpec((T_BLK, B, C), lambda i: (i, 0, 0)),
            pl.BlockSpec((B, SP), lambda i: (0, 0)),
            pl.BlockSpec((B, SP), lambda i: (0, 0)),
            pl.BlockSpec((C, C), lambda i: (0, 0)),
            pl.BlockSpec((B, SP), lambda i: (0, 0)),
        ],
        out_specs=[
            pl.BlockSpec((1, 1), lambda i: (0, 0)),
            pl.BlockSpec((1, 1), lambda i: (0, 0)),
            pl.BlockSpec((1, 1), lambda i: (0, 0)),
        ],
        out_shape=[
            jax.ShapeDtypeStruct((1, 1), jnp.float32),
            jax.ShapeDtypeStruct((1, 1), jnp.float32),
            jax.ShapeDtypeStruct((1, 1), jnp.float32),
        ],
        scratch_shapes=[
            pltpu.VMEM((C, C), jnp.bfloat16),
            pltpu.VMEM((B, C, SP), jnp.bfloat16),
            pltpu.VMEM((T_BLK, B, C), jnp.float32),
            pltpu.VMEM((T_BLK, B, SP), jnp.float32),
            pltpu.VMEM((B, C), jnp.float32),
            pltpu.VMEM((B, 1), jnp.float32),
            pltpu.VMEM((B, SP), jnp.float32),
        ],
        compiler_params=pltpu.CompilerParams(
            dimension_semantics=("arbitrary",),
        ),
    )(xt, ext, skip, den_trans, frames_b)

    return out_s[0, 0], out_tf[0, 0], out_af[0, 0]


# speedup vs baseline: 1.0052x; 1.0001x over previous
import jax
import jax.numpy as jnp
from jax.experimental import pallas as pl
from jax.experimental.pallas import tpu as pltpu

B, T, C, L = 32, 500, 256, 50
S = 2 * L + 1
SP = 128
T_BLK = 100
NT = T // T_BLK
DEN_SCALE = 1.0
NEG = -1e30


def _fwd_kernel(x_ref, ext_ref, skip_ref, dent_ref, frames_ref,
                out_s_ref, out_tf_ref, out_af_ref,
                p_ref, oh_ref, probs_ref, pemit_ref,
                z_ref, lz_ref, na_ref):
    i = pl.program_id(0)

    @pl.when(i == 0)
    def _init():
        dt = dent_ref[...]
        m = jnp.max(dt, axis=1, keepdims=True)
        e = jnp.exp(dt - m)
        p_ref[...] = (e / jnp.sum(e, axis=1, keepdims=True)).astype(jnp.bfloat16)
        ids = jax.lax.broadcasted_iota(jnp.int32, (B, C, SP), 1)
        oh_ref[...] = (ids == ext_ref[...][:, None, :]).astype(jnp.bfloat16)
        z_ref[...] = jnp.zeros((B, C), jnp.float32)
        lz_ref[...] = jnp.zeros((B, 1), jnp.float32)
        lane = jax.lax.broadcasted_iota(jnp.int32, (B, SP), 1)
        na_ref[...] = jnp.where(lane == 0, 0.0, NEG)

    x = x_ref[...]
    m = jnp.max(x, axis=2, keepdims=True)
    e = jnp.exp(x - m)
    sm = jnp.sum(e, axis=2, keepdims=True)
    lp = x - m - jnp.log(sm)
    probs_ref[...] = e / sm

    lp16 = lp.astype(jnp.bfloat16)
    for b in range(B):
        pemit_ref[:, b, :] = jnp.dot(lp16[:, b, :], oh_ref[b],
                                     preferred_element_type=jnp.float32)

    P = p_ref[...]
    skip = skip_ref[...] > 0

    NORM = 10
    z = z_ref[...]
    lz = lz_ref[...]
    na = na_ref[...]
    for t in range(T_BLK):
        pt = probs_ref[t]
        et = pemit_ref[t]
        u = jnp.dot(z.astype(jnp.bfloat16), P,
                    preferred_element_type=jnp.float32) * pt
        if t == 0:
            u = jnp.where(i == 0, pt * (1.0 / C), u)
        if t % NORM == NORM - 1:
            s = jnp.sum(u, axis=1, keepdims=True)
            z = u * (1.0 / s)
            lz = lz + jnp.log(s)
        else:
            z = u
        a1 = jnp.concatenate([jnp.full((B, 1), NEG, jnp.float32), na[:, :-1]], axis=1)
        a2 = jnp.where(skip,
                       jnp.concatenate([jnp.full((B, 2), NEG, jnp.float32), na[:, :-2]], axis=1),
                       jnp.float32(NEG))
        mx = jnp.maximum(jnp.maximum(na, a1), a2)
        na = mx + jnp.log(jnp.exp(na - mx) + jnp.exp(a1 - mx) + jnp.exp(a2 - mx)) + et
    z_ref[...] = z
    lz_ref[...] = lz
    na_ref[...] = na

    @pl.when(i == NT - 1)
    def _finish():
        aL, aK = na[:, S - 1:S], na[:, S - 2:S - 1]
        m2 = jnp.maximum(aL, aK)
        num_tot = m2 + jnp.log(jnp.exp(aL - m2) + jnp.exp(aK - m2))
        den_tot = lz
        ts = num_tot - DEN_SCALE * den_tot
        fr = frames_ref[...][:, 0:1]
        mask = jnp.isfinite(ts) & (ts > -1e20)
        out_s_ref[...] = jnp.sum(jnp.where(mask, ts, 0.0)).reshape(1, 1)
        out_tf_ref[...] = jnp.sum(jnp.where(mask, fr, 0.0)).reshape(1, 1)
        out_af_ref[...] = jnp.sum(fr).reshape(1, 1)


@jax.jit
def kernel(nnet_output, labels, supervision_segments, den_trans):
    ext = jnp.full((B, SP), -1, jnp.int32)
    ext = ext.at[:, 0:S:2].set(0)
    ext = ext.at[:, 1:S:2].set(labels)
    ext_prev2 = jnp.concatenate([jnp.full((B, 2), -1, jnp.int32), ext[:, :-2]], axis=1)
    skip = ((ext > 0) & (ext != ext_prev2)).astype(jnp.float32)
    frames = supervision_segments[:, 2].astype(jnp.float32)
    frames_b = jnp.broadcast_to(frames[:, None], (B, SP))

    xt = jnp.transpose(nnet_output, (1, 0, 2))

    grid = (NT,)
    out_s, out_tf, out_af = pl.pallas_call(
        _fwd_kernel,
        grid=grid,
        in_specs=[
            pl.BlockSpec((T_BLK, B, C), lambda i: (i, 0, 0)),
            pl.BlockSpec((B, SP), lambda i: (0, 0)),
            pl.BlockSpec((B, SP), lambda i: (0, 0)),
            pl.BlockSpec((C, C), lambda i: (0, 0)),
            pl.BlockSpec((B, SP), lambda i: (0, 0)),
        ],
        out_specs=[
            pl.BlockSpec((1, 1), lambda i: (0, 0)),
            pl.BlockSpec((1, 1), lambda i: (0, 0)),
            pl.BlockSpec((1, 1), lambda i: (0, 0)),
        ],
        out_shape=[
            jax.ShapeDtypeStruct((1, 1), jnp.float32),
            jax.ShapeDtypeStruct((1, 1), jnp.float32),
            jax.ShapeDtypeStruct((1, 1), jnp.float32),
        ],
        scratch_shapes=[
            pltpu.VMEM((C, C), jnp.bfloat16),
            pltpu.VMEM((B, C, SP), jnp.bfloat16),
            pltpu.VMEM((T_BLK, B, C), jnp.float32),
            pltpu.VMEM((T_BLK, B, SP), jnp.float32),
            pltpu.VMEM((B, C), jnp.float32),
            pltpu.VMEM((B, 1), jnp.float32),
            pltpu.VMEM((B, SP), jnp.float32),
        ],
        compiler_params=pltpu.CompilerParams(
            dimension_semantics=("arbitrary",),
        ),
    )(xt, ext, skip, den_trans, frames_b)

    return out_s[0, 0], out_tf[0, 0], out_af[0, 0]
